# baseline (speedup 1.0000x reference)
import functools

import jax
import jax.numpy as jnp
from jax import lax
from jax.experimental import pallas as pl
from jax.experimental.pallas import tpu as pltpu
from jax.experimental.pallas import tpu_sc as plsc

_D = 64
_B = 16384
_L = 16
_NW = 32
_BPW = _B // _NW
_C = 128
_NCHUNK = _BPW // _C
_NG = _C // _L
_SCR = 2 * _B * _D


def _make_imag_fetch():
    mesh = plsc.VectorSubcoreMesh(core_axis_name="c", subcore_axis_name="s")

    @functools.partial(
        pl.kernel,
        mesh=mesh,
        out_type=jax.ShapeDtypeStruct((_SCR,), jnp.float32),
        scratch_types=[
            pltpu.VMEM((_BPW,), jnp.int32),
            pltpu.VMEM((_BPW,), jnp.int32),
            pltpu.VMEM((_C, _D), jnp.float32),
            pltpu.VMEM((_C, _D), jnp.float32),
            pltpu.VMEM((_C, _D), jnp.float32),
            pltpu.VMEM((_C, _D), jnp.float32),
            pltpu.VMEM((_C * _D,), jnp.float32),
            pltpu.SemaphoreType.DMA,
            pltpu.SemaphoreType.DMA,
        ],
        compiler_params=pltpu.CompilerParams(use_tc_tiling_on_sc=True),
    )
    def imag_fetch(heads_hbm, tails_hbm, ei_hbm, scr_hbm,
                   idx_h, idx_t, hi0, ti0, hi1, ti1, flat_v, sem0, sem1):
        wid = lax.axis_index("s") * 2 + lax.axis_index("c")
        base = pl.multiple_of(wid * _BPW, _BPW)
        pltpu.sync_copy(heads_hbm.at[pl.ds(base, _BPW)], idx_h)
        pltpu.sync_copy(tails_hbm.at[pl.ds(base, _BPW)], idx_t)
        slots = ((hi0, ti0, sem0), (hi1, ti1, sem1))

        def issue(c, slot):
            hi_v, ti_v, sem = slot

            def issue_body(g, carry):
                hvec = idx_h[pl.ds(c * _C + g * _L, _L)]
                tvec = idx_t[pl.ds(c * _C + g * _L, _L)]
                for k in range(_L):
                    i = g * _L + k
                    pltpu.async_copy(ei_hbm.at[hvec[k]], hi_v.at[i], sem)
                    pltpu.async_copy(ei_hbm.at[tvec[k]], ti_v.at[i], sem)
                return carry

            lax.fori_loop(0, _NG, issue_body, 0)

        def flush(c, slot):
            hi_v, ti_v, sem = slot
            for buf in (hi_v, ti_v):
                pltpu.make_async_copy(ei_hbm.at[pl.ds(0, _C)], buf,
                                      sem).wait()
            for buf, off in ((hi_v, base * _D + c * _C * _D),
                             (ti_v, _B * _D + base * _D + c * _C * _D)):
                def flat_body(i, carry2, _buf=buf):
                    for j in range(_D // _L):
                        flat_v[pl.ds(i * _D + j * _L, _L)] = (
                            _buf[i, pl.ds(j * _L, _L)])
                    return carry2
                lax.fori_loop(0, _C, flat_body, 0)
                pltpu.sync_copy(flat_v, scr_hbm.at[pl.ds(off, _C * _D)])

        issue(0, slots[0])
        for c in range(_NCHUNK):
            if c + 1 < _NCHUNK:
                issue(c + 1, slots[(c + 1) % 2])
            flush(c, slots[c % 2])

    return imag_fetch


def _make_score():
    mesh = plsc.VectorSubcoreMesh(core_axis_name="c", subcore_axis_name="s")

    @functools.partial(
        pl.kernel,
        mesh=mesh,
        out_type=jax.ShapeDtypeStruct((_B,), jnp.float32),
        scratch_types=[
            pltpu.VMEM((_C,), jnp.int32),
            pltpu.VMEM((_C,), jnp.int32),
            pltpu.VMEM((_C,), jnp.int32),
            pltpu.VMEM((_C, _D), jnp.float32),
            pltpu.VMEM((_C, _D), jnp.float32),
            pltpu.VMEM((_C, _D), jnp.float32),
            pltpu.VMEM((_C, _D), jnp.float32),
            pltpu.VMEM((_C * _D,), jnp.float32),
            pltpu.VMEM((_C * _D,), jnp.float32),
            pltpu.VMEM((_BPW,), jnp.float32),
            pltpu.SemaphoreType.DMA,
        ],
        compiler_params=pltpu.CompilerParams(use_tc_tiling_on_sc=False),
    )
    def score(heads_hbm, rels_hbm, tails_hbm, er_hbm, rr_hbm, ri_hbm,
              scr_hbm, out_hbm,
              idx_h, idx_t, idx_r, hr_v, tr_v, rr_v, ri_v, hi_v, ti_v,
              out_v, sem):
        wid = lax.axis_index("s") * 2 + lax.axis_index("c")
        base = pl.multiple_of(wid * _BPW, _BPW)
        lanes = lax.iota(jnp.int32, _L)
        perms = [lanes ^ sh for sh in (8, 4, 2, 1)]
        lane_masks = [lanes == k for k in range(_L)]

        def chunk_body(c, carry):
            cbase = base + c * _C
            pltpu.sync_copy(heads_hbm.at[pl.ds(cbase, _C)], idx_h)
            pltpu.sync_copy(tails_hbm.at[pl.ds(cbase, _C)], idx_t)
            pltpu.sync_copy(rels_hbm.at[pl.ds(cbase, _C)], idx_r)

            cps = [
                pltpu.async_copy(er_hbm.at[idx_h], hr_v, sem),
                pltpu.async_copy(er_hbm.at[idx_t], tr_v, sem),
                pltpu.async_copy(rr_hbm.at[idx_r], rr_v, sem),
                pltpu.async_copy(ri_hbm.at[idx_r], ri_v, sem),
                pltpu.async_copy(scr_hbm.at[pl.ds(cbase * _D, _C * _D)],
                                 hi_v, sem),
                pltpu.async_copy(scr_hbm.at[pl.ds(_B * _D + cbase * _D,
                                                  _C * _D)], ti_v, sem),
            ]
            for cp in cps:
                cp.wait()

            def group_body(g, carry2):
                out_vec = jnp.zeros((_L,), jnp.float32)
                for k in range(_L):
                    i = g * _L + k
                    acc = jnp.zeros((_L,), jnp.float32)
                    for j in range(_D // _L):
                        sl = pl.ds(j * _L, _L)
                        fl = pl.ds(i * _D + j * _L, _L)
                        hr = hr_v[i, sl]
                        tr = tr_v[i, sl]
                        hi = hi_v[fl]
                        ti = ti_v[fl]
                        s = rr_v[i, sl] + ri_v[i, sl]
                        acc = acc + s * ((hr - hi) * tr + (hr + hi) * ti)
                    for perm in perms:
                        acc = acc + _lane_shuffle(acc, perm)
                    out_vec = lax.select(lane_masks[k], acc, out_vec)
                out_v[pl.ds(c * _C + g * _L, _L)] = out_vec
                return carry2

            lax.fori_loop(0, _NG, group_body, 0)
            return carry

        lax.fori_loop(0, _NCHUNK, chunk_body, 0)
        pltpu.sync_copy(out_v, out_hbm.at[pl.ds(base, _BPW)])

    return score


_GATHER_DNUMS = lax.GatherDimensionNumbers(
    offset_dims=(), collapsed_slice_dims=(0,), start_index_map=(0,))


def _lane_shuffle(v, perm):
    return lax.gather(v, perm[:, None], _GATHER_DNUMS, slice_sizes=(1,),
                      mode=lax.GatherScatterMode.PROMISE_IN_BOUNDS)


_imag_fetch = _make_imag_fetch()
_score = _make_score()


def kernel(heads, relations, tails, entity_real, entity_imag,
           relation_real, relation_imag):
    heads = heads.astype(jnp.int32)
    tails = tails.astype(jnp.int32)
    relations = relations.astype(jnp.int32)
    scr = _imag_fetch(heads, tails, entity_imag)
    return _score(heads, relations, tails, entity_real,
                  relation_real, relation_imag, scr)


# speedup vs baseline: 1.3179x; 1.3179x over previous
import functools

import jax
import jax.numpy as jnp
from jax import lax
from jax.experimental import pallas as pl
from jax.experimental.pallas import tpu as pltpu
from jax.experimental.pallas import tpu_sc as plsc

_N_ENT = 1000000
_N_REL = 1000
_D = 64
_B = 16384
_L = 16
_NW = 32
_BPW = _B // _NW
_C = 128
_NCHUNK = _BPW // _C
_NG = _C // _L
_EB = 2048


def _combine_entities_body(ert_ref, eit_ref, c_ref):
    c_ref[:, 0:_D] = jnp.transpose(ert_ref[...])
    c_ref[:, _D:2 * _D] = jnp.transpose(eit_ref[...])


_combine_entities = pl.pallas_call(
    _combine_entities_body,
    grid=(pl.cdiv(_N_ENT, _EB),),
    in_specs=[
        pl.BlockSpec((_D, _EB), lambda g: (0, g)),
        pl.BlockSpec((_D, _EB), lambda g: (0, g)),
    ],
    out_specs=pl.BlockSpec((_EB, 2 * _D), lambda g: (g, 0)),
    out_shape=jax.ShapeDtypeStruct((_N_ENT, 2 * _D), jnp.float32),
)


def _combine_relations_body(rrt_ref, rit_ref, s_ref):
    s_ref[...] = jnp.zeros_like(s_ref)
    s_ref[:, 0:_D] = jnp.transpose(rrt_ref[...] + rit_ref[...])


_combine_relations = pl.pallas_call(
    _combine_relations_body,
    in_specs=[
        pl.BlockSpec((_D, _N_REL), lambda: (0, 0)),
        pl.BlockSpec((_D, _N_REL), lambda: (0, 0)),
    ],
    out_specs=pl.BlockSpec((_N_REL, 2 * _D), lambda: (0, 0)),
    out_shape=jax.ShapeDtypeStruct((_N_REL, 2 * _D), jnp.float32),
)


def _make_score():
    mesh = plsc.VectorSubcoreMesh(core_axis_name="c", subcore_axis_name="s")

    @functools.partial(
        pl.kernel,
        mesh=mesh,
        out_type=jax.ShapeDtypeStruct((_B,), jnp.float32),
        scratch_types=[
            pltpu.VMEM((_C,), jnp.int32),
            pltpu.VMEM((_C,), jnp.int32),
            pltpu.VMEM((_C,), jnp.int32),
            pltpu.VMEM((_C, 2 * _D), jnp.float32),
            pltpu.VMEM((_C, 2 * _D), jnp.float32),
            pltpu.VMEM((_C, 2 * _D), jnp.float32),
            pltpu.VMEM((_BPW,), jnp.float32),
            pltpu.SemaphoreType.DMA,
        ],
        compiler_params=pltpu.CompilerParams(use_tc_tiling_on_sc=True),
    )
    def score(heads_hbm, rels_hbm, tails_hbm, c_hbm, s_hbm, out_hbm,
              idx_h, idx_t, idx_r, h_v, t_v, s_v, out_v, sem):
        wid = lax.axis_index("s") * 2 + lax.axis_index("c")
        base = pl.multiple_of(wid * _BPW, _BPW)
        lanes = lax.iota(jnp.int32, _L)
        perms = [lanes ^ sh for sh in (8, 4, 2, 1)]
        lane_masks = [lanes == k for k in range(_L)]

        def chunk_body(c, carry):
            cbase = base + c * _C
            pltpu.sync_copy(heads_hbm.at[pl.ds(cbase, _C)], idx_h)
            pltpu.sync_copy(tails_hbm.at[pl.ds(cbase, _C)], idx_t)
            pltpu.sync_copy(rels_hbm.at[pl.ds(cbase, _C)], idx_r)

            cps = [
                pltpu.async_copy(c_hbm.at[idx_h], h_v, sem),
                pltpu.async_copy(c_hbm.at[idx_t], t_v, sem),
                pltpu.async_copy(s_hbm.at[idx_r], s_v, sem),
            ]
            for cp in cps:
                cp.wait()

            def group_body(g, carry2):
                out_vec = jnp.zeros((_L,), jnp.float32)
                for k in range(_L):
                    i = g * _L + k
                    acc = jnp.zeros((_L,), jnp.float32)
                    for j in range(_D // _L):
                        sl = pl.ds(j * _L, _L)
                        sl2 = pl.ds(_D + j * _L, _L)
                        hr = h_v[i, sl]
                        hi = h_v[i, sl2]
                        tr = t_v[i, sl]
                        ti = t_v[i, sl2]
                        s = s_v[i, sl]
                        acc = acc + s * ((hr - hi) * tr + (hr + hi) * ti)
                    for perm in perms:
                        acc = acc + _lane_shuffle(acc, perm)
                    out_vec = lax.select(lane_masks[k], acc, out_vec)
                out_v[pl.ds(c * _C + g * _L, _L)] = out_vec
                return carry2

            lax.fori_loop(0, _NG, group_body, 0)
            return carry

        lax.fori_loop(0, _NCHUNK, chunk_body, 0)
        pltpu.sync_copy(out_v, out_hbm.at[pl.ds(base, _BPW)])

    return score


_GATHER_DNUMS = lax.GatherDimensionNumbers(
    offset_dims=(), collapsed_slice_dims=(0,), start_index_map=(0,))


def _lane_shuffle(v, perm):
    return lax.gather(v, perm[:, None], _GATHER_DNUMS, slice_sizes=(1,),
                      mode=lax.GatherScatterMode.PROMISE_IN_BOUNDS)


_score = _make_score()


def kernel(heads, relations, tails, entity_real, entity_imag,
           relation_real, relation_imag):
    comb = _combine_entities(entity_real.T, entity_imag.T)
    srel = _combine_relations(relation_real.T, relation_imag.T)
    return _score(heads.astype(jnp.int32), relations.astype(jnp.int32),
                  tails.astype(jnp.int32), comb, srel)
